# baseline (speedup 1.0000x reference)
import functools

import jax
import jax.numpy as jnp
from jax import lax
from jax.experimental import pallas as pl
from jax.experimental.pallas import tpu as pltpu
from jax.experimental.pallas import tpu_sc as plsc

N_NODES = 10000
N_EDGES = 320000
D = 128
NPAD = 10240
NUM_GRAPHS = 8
NEG = 0.01
NC = 2
NS = 16
NT = NC * NS
CH = 128
TOT_CH = 2560
NCHQ = TOT_CH // NT
NCHH = TOT_CH // NS
NBUF = 4
E_PAD = TOT_CH * CH
ROWS_TILE = NPAD // NS
NQ = 4
DQ = D // NQ
DH = D // 2
ZR = 128
BLK = 1024
GRID = NPAD // BLK


CNT_BYTES = CH * 16 * 4


def _pipeline(h_hbm, src_v, dst_v, rbuf, gsem, ssem, acc_sp, nch, nbytes,
              cnt_sp=None, ones_v=None, csem=None):
    ngrp = nch // NBUF
    for p in range(NBUF):
        pltpu.async_copy(h_hbm.at[src_v.at[p]], rbuf[p], gsem[p])

    def grp(i, carry):
        sdescs, cdescs = [], []
        for p in range(NBUF):
            j = i * NBUF + p
            pltpu.make_async_copy(
                h_hbm.at[pl.ds(0, CH)], rbuf[p], gsem[p]).wait()
            sdescs.append(pltpu.async_copy(
                rbuf[p], acc_sp.at[dst_v.at[j]], ssem[p], add=True))
            if cnt_sp is not None:
                cdescs.append(pltpu.async_copy(
                    ones_v, cnt_sp.at[dst_v.at[j]], csem[p], add=True))
        for p in range(NBUF):
            sdescs[p].wait()
            if cnt_sp is not None:
                cdescs[p].wait()
            pltpu.async_copy(h_hbm.at[src_v.at[(i + 1) * NBUF + p]],
                             rbuf[p], gsem[p])
        return carry

    lax.fori_loop(0, ngrp - 1, grp, 0)
    sdescs, cdescs = [], []
    for p in range(NBUF):
        j = (ngrp - 1) * NBUF + p
        pltpu.make_async_copy(
            h_hbm.at[pl.ds(0, CH)], rbuf[p], gsem[p]).wait()
        sdescs.append(pltpu.async_copy(
            rbuf[p], acc_sp.at[dst_v.at[j]], ssem[p], add=True))
        if cnt_sp is not None:
            cdescs.append(pltpu.async_copy(
                ones_v, cnt_sp.at[dst_v.at[j]], csem[p], add=True))
    for p in range(NBUF):
        sdescs[p].wait()
        if cnt_sp is not None:
            cdescs[p].wait()


def _sc_agg0_body(*refs):
    (h0, h1, h2, h3, src2_hbm, dst2_hbm, ones_hbm, zrow_hbm, zcnt_hbm,
     o0, o1, o2, o3, cnt_hbm,
     src_v, dst_v, r0, r1, r2, r3, ones_v, stage_v, zstage_v, czstage_v,
     cstage_v, acc_sp, cnt_sp,
     g0, g1, g2, g3, s0, s1, s2, s3, c0, c1, c2, c3) = refs
    hq = (h0, h1, h2, h3)
    oq = (o0, o1, o2, o3)
    rbuf = (r0, r1, r2, r3)
    gsem = (g0, g1, g2, g3)
    ssem = (s0, s1, s2, s3)
    csem = (c0, c1, c2, c3)
    cid = lax.axis_index("c")
    sid = lax.axis_index("s")
    wid = cid * NS + sid
    tbase = sid * ROWS_TILE
    obase = cid * NPAD + tbase
    nz = ROWS_TILE // ZR
    pltpu.sync_copy(zrow_hbm, zstage_v)
    pltpu.sync_copy(ones_hbm, ones_v)
    pltpu.sync_copy(zcnt_hbm, czstage_v)
    pltpu.sync_copy(src2_hbm.at[pl.ds(wid * NCHQ, NCHQ)], src_v)
    pltpu.sync_copy(dst2_hbm.at[pl.ds(wid * NCHQ, NCHQ)], dst_v)
    for q in range(NQ):
        counts = q == 0
        for j in range(nz):
            pltpu.sync_copy(zstage_v, acc_sp.at[pl.ds(tbase + j * ZR, ZR)])
        if counts:
            for j in range(nz):
                pltpu.sync_copy(czstage_v,
                                cnt_sp.at[pl.ds(tbase + j * ZR, ZR)])
        plsc.subcore_barrier()
        _pipeline(hq[q], src_v, dst_v, rbuf, gsem, ssem, acc_sp, NCHQ,
                  CH * DQ * 4, cnt_sp if counts else None, ones_v, csem)
        plsc.subcore_barrier()
        for j in range(nz):
            pltpu.sync_copy(acc_sp.at[pl.ds(tbase + j * ZR, ZR)], stage_v)
            pltpu.sync_copy(stage_v, oq[q].at[pl.ds(obase + j * ZR, ZR)])
        if counts:
            for j in range(nz):
                pltpu.sync_copy(cnt_sp.at[pl.ds(tbase + j * ZR, ZR)],
                                cstage_v)
                pltpu.sync_copy(cstage_v,
                                cnt_hbm.at[pl.ds(obase + j * ZR, ZR)])


def _sc_agg_cs_body(*refs):
    (hh0, hh1, src2_hbm, dst2_hbm, zrow_hbm,
     out0, out1,
     src_v, dst_v, r0, r1, r2, r3, stage_v, zstage_v, acc_sp,
     g0, g1, g2, g3, s0, s1, s2, s3) = refs
    rbuf = (r0, r1, r2, r3)
    gsem = (g0, g1, g2, g3)
    ssem = (s0, s1, s2, s3)
    cid = lax.axis_index("c")
    sid = lax.axis_index("s")
    tbase = sid * ROWS_TILE
    nz = ROWS_TILE // ZR
    pltpu.sync_copy(zrow_hbm, zstage_v)
    pltpu.sync_copy(src2_hbm.at[pl.ds(sid * NCHH, NCHH)], src_v)
    pltpu.sync_copy(dst2_hbm.at[pl.ds(sid * NCHH, NCHH)], dst_v)
    for j in range(nz):
        pltpu.sync_copy(zstage_v, acc_sp.at[pl.ds(tbase + j * ZR, ZR)])
    plsc.subcore_barrier()

    @pl.when(cid == 0)
    def _():
        _pipeline(hh0, src_v, dst_v, rbuf, gsem, ssem, acc_sp, NCHH,
                  CH * DH * 4)

    @pl.when(cid == 1)
    def _():
        _pipeline(hh1, src_v, dst_v, rbuf, gsem, ssem, acc_sp, NCHH,
                  CH * DH * 4)

    plsc.subcore_barrier()

    @pl.when(cid == 0)
    def _():
        for j in range(nz):
            pltpu.sync_copy(acc_sp.at[pl.ds(tbase + j * ZR, ZR)], stage_v)
            pltpu.sync_copy(stage_v, out0.at[pl.ds(tbase + j * ZR, ZR)])

    @pl.when(cid == 1)
    def _():
        for j in range(nz):
            pltpu.sync_copy(acc_sp.at[pl.ds(tbase + j * ZR, ZR)], stage_v)
            pltpu.sync_copy(stage_v, out1.at[pl.ds(tbase + j * ZR, ZR)])


@functools.cache
def _get_sc_kernels():
    mesh = plsc.VectorSubcoreMesh(core_axis_name="c", subcore_axis_name="s",
                                  num_cores=NC, num_subcores=NS)
    qout = [jax.ShapeDtypeStruct((NC * NPAD, DQ), jnp.float32)
            for _ in range(NQ)]
    cparams = pltpu.CompilerParams(use_tc_tiling_on_sc=False)
    sc_agg0 = pl.kernel(
        _sc_agg0_body,
        out_type=qout + [jax.ShapeDtypeStruct((NC * NPAD, 16), jnp.float32)],
        compiler_params=cparams,
        mesh=mesh,
        scratch_types=[pltpu.VMEM((NCHQ, CH), jnp.int32),
                       pltpu.VMEM((NCHQ, CH), jnp.int32)]
                      + [pltpu.VMEM((CH, DQ), jnp.float32)
                         for _ in range(NBUF)]
                      + [pltpu.VMEM((CH, 16), jnp.float32),
                         pltpu.VMEM((ZR, DQ), jnp.float32),
                         pltpu.VMEM((ZR, DQ), jnp.float32),
                         pltpu.VMEM((ZR, 16), jnp.float32),
                         pltpu.VMEM((ZR, 16), jnp.float32),
                         pltpu.VMEM_SHARED((NPAD, DQ), jnp.float32),
                         pltpu.VMEM_SHARED((NPAD, 16), jnp.float32)]
                      + [pltpu.SemaphoreType.DMA for _ in range(3 * NBUF)],
    )
    sc_agg_cs = pl.kernel(
        _sc_agg_cs_body,
        out_type=[jax.ShapeDtypeStruct((NPAD, DH), jnp.float32),
                  jax.ShapeDtypeStruct((NPAD, DH), jnp.float32)],
        compiler_params=cparams,
        mesh=mesh,
        scratch_types=[pltpu.VMEM((NCHH, CH), jnp.int32),
                       pltpu.VMEM((NCHH, CH), jnp.int32)]
                      + [pltpu.VMEM((CH, DH), jnp.float32)
                         for _ in range(NBUF)]
                      + [pltpu.VMEM((ZR, DH), jnp.float32),
                         pltpu.VMEM((ZR, DH), jnp.float32),
                         pltpu.VMEM_SHARED((NPAD, DH), jnp.float32)]
                      + [pltpu.SemaphoreType.DMA for _ in range(2 * NBUF)],
    )
    return sc_agg0, sc_agg_cs


def _sc_aggregate0(hqs, src2, dst2, ones16, zrow, zcnt):
    return _get_sc_kernels()[0](*hqs, src2, dst2, ones16, zrow, zcnt)


def _sc_aggregate_cs(hh0, hh1, src2, dst2, zrow64):
    return _get_sc_kernels()[1](hh0, hh1, src2, dst2, zrow64)


def _tc_inv_body(cnt_ref, inv_ref):
    c = cnt_ref[0, :, 0:1] + cnt_ref[1, :, 0:1]
    inv = 1.0 / jnp.maximum(c, 1.0)
    inv_ref[...] = jnp.broadcast_to(inv, (BLK, D))


def _tc_inv(cnt):
    return pl.pallas_call(
        _tc_inv_body,
        grid=(GRID,),
        in_specs=[pl.BlockSpec((2, BLK, 16), lambda i: (0, i, 0))],
        out_specs=pl.BlockSpec((BLK, D), lambda i: (i, 0)),
        out_shape=jax.ShapeDtypeStruct((NPAD, D), jnp.float32),
    )(cnt)


def _tc_layer_body(p0_ref, p1_ref, p2_ref, p3_ref, inv_ref, h_ref,
                   wlT_ref, bl_ref, wrT_ref, out_ref):
    agg = jnp.concatenate(
        [p[0] + p[1] for p in (p0_ref, p1_ref, p2_ref, p3_ref)],
        axis=-1) * inv_ref[...]
    o = (jnp.dot(agg, wlT_ref[...], preferred_element_type=jnp.float32)
         + jnp.dot(h_ref[...], wrT_ref[...], preferred_element_type=jnp.float32)
         + bl_ref[...])
    out_ref[...] = jnp.where(o >= 0, o, NEG * o)


def _tc_layer(parts, inv, h, wlT, bl, wrT):
    qspec = pl.BlockSpec((2, BLK, DQ), lambda i: (0, i, 0))
    return pl.pallas_call(
        _tc_layer_body,
        grid=(GRID,),
        in_specs=[qspec, qspec, qspec, qspec,
                  pl.BlockSpec((BLK, D), lambda i: (i, 0)),
                  pl.BlockSpec((BLK, D), lambda i: (i, 0)),
                  pl.BlockSpec((D, D), lambda i: (0, 0)),
                  pl.BlockSpec((1, D), lambda i: (0, 0)),
                  pl.BlockSpec((D, D), lambda i: (0, 0))],
        out_specs=pl.BlockSpec((BLK, D), lambda i: (i, 0)),
        out_shape=jax.ShapeDtypeStruct((NPAD, D), jnp.float32),
    )(*parts, inv, h, wlT, bl, wrT)


def _tc_layer_cs_body(o0_ref, o1_ref, inv_ref, h_ref,
                      wlT_ref, bl_ref, wrT_ref, out_ref):
    agg = jnp.concatenate([o0_ref[...], o1_ref[...]], axis=-1) * inv_ref[...]
    o = (jnp.dot(agg, wlT_ref[...], preferred_element_type=jnp.float32)
         + jnp.dot(h_ref[...], wrT_ref[...], preferred_element_type=jnp.float32)
         + bl_ref[...])
    out_ref[...] = jnp.where(o >= 0, o, NEG * o)


def _tc_layer_cs(o0, o1, inv, h, wlT, bl, wrT):
    hspec = pl.BlockSpec((BLK, DH), lambda i: (i, 0))
    return pl.pallas_call(
        _tc_layer_cs_body,
        grid=(GRID,),
        in_specs=[hspec, hspec,
                  pl.BlockSpec((BLK, D), lambda i: (i, 0)),
                  pl.BlockSpec((BLK, D), lambda i: (i, 0)),
                  pl.BlockSpec((D, D), lambda i: (0, 0)),
                  pl.BlockSpec((1, D), lambda i: (0, 0)),
                  pl.BlockSpec((D, D), lambda i: (0, 0))],
        out_specs=pl.BlockSpec((BLK, D), lambda i: (i, 0)),
        out_shape=jax.ShapeDtypeStruct((NPAD, D), jnp.float32),
    )(o0, o1, inv, h, wlT, bl, wrT)


def _mempool_body(h_ref, bt_ref, k1T_ref, conv_ref, w1T_ref, b1_ref,
                  w2T_ref, b2_ref, out_ref, acc_ref):
    i = pl.program_id(0)
    h = h_ref[...]
    k1T = k1T_ref[...]
    xk = jnp.dot(h, k1T, preferred_element_type=jnp.float32)
    x2 = jnp.sum(h * h, axis=1, keepdims=True)
    k2 = jnp.sum(k1T * k1T, axis=0, keepdims=True)
    col = lax.broadcasted_iota(jnp.int32, (BLK, D), 1)
    dist = jnp.maximum(x2 + k2 - 2.0 * xk, 0.0)
    t = 1.0 / (1.0 + dist)
    r = lax.broadcasted_iota(jnp.int32, (D, D), 0)
    c2 = lax.broadcasted_iota(jnp.int32, (D, D), 1)
    G = jnp.where((r // 10 == c2 // 10) & (r < 40) & (c2 < 40), 1.0, 0.0)
    gsum = jnp.dot(t, G, preferred_element_type=jnp.float32)
    u = jnp.where(col < 40, (t / gsum) * conv_ref[...], 0.0)
    P = jnp.where((r % 10 == c2) & (r < 40) & (c2 < 10), 1.0, 0.0)
    logits = jnp.dot(u, P, preferred_element_type=jnp.float32)
    neg = jnp.float32(-1e30)
    lm = jnp.max(jnp.where(col < 10, logits, neg), axis=1, keepdims=True)
    e = jnp.where(col < 10, jnp.exp(logits - lm), 0.0)
    S = e / jnp.sum(e, axis=1, keepdims=True)
    T = jnp.where((r < 10) & (r == c2 % 10), 1.0, 0.0)
    W = jnp.dot(S, T, preferred_element_type=jnp.float32) * bt_ref[...]
    contrib = lax.dot_general(W, h, (((0,), (0,)), ((), ())),
                              preferred_element_type=jnp.float32)

    @pl.when(i == 0)
    def _():
        acc_ref[...] = contrib

    @pl.when(i > 0)
    def _():
        acc_ref[...] += contrib

    @pl.when(i == GRID - 1)
    def _():
        acc = acc_ref[...]
        xp = jnp.dot(acc, w1T_ref[...], preferred_element_type=jnp.float32) + b1_ref[...]
        xp = jnp.where(xp >= 0, xp, NEG * xp)
        Q = jnp.where((c2 // 10 == r) & (r < 8) & (c2 < 80), 1.0, 0.0)
        ypre = jnp.dot(Q, xp, preferred_element_type=jnp.float32)
        out_ref[...] = (jnp.dot(ypre, w2T_ref[...], preferred_element_type=jnp.float32)
                        + b2_ref[...])


def _tc_mempool(h, btiled, k1T, convrow, w1T, b1, w2T, b2):
    return pl.pallas_call(
        _mempool_body,
        grid=(GRID,),
        in_specs=[pl.BlockSpec((BLK, D), lambda i: (i, 0)),
                  pl.BlockSpec((BLK, D), lambda i: (i, 0)),
                  pl.BlockSpec((D, D), lambda i: (0, 0)),
                  pl.BlockSpec((1, D), lambda i: (0, 0)),
                  pl.BlockSpec((D, D), lambda i: (0, 0)),
                  pl.BlockSpec((1, D), lambda i: (0, 0)),
                  pl.BlockSpec((D, D), lambda i: (0, 0)),
                  pl.BlockSpec((1, D), lambda i: (0, 0))],
        out_specs=pl.BlockSpec((D, D), lambda i: (0, 0)),
        out_shape=jax.ShapeDtypeStruct((D, D), jnp.float32),
        scratch_shapes=[pltpu.VMEM((D, D), jnp.float32)],
    )(h, btiled, k1T, convrow, w1T, b1, w2T, b2)


def kernel(x, edge_index, batch, W_l0, b_l0, W_r0, W_l1, b_l1, W_r1,
           W_l2, b_l2, W_r2, k1, conv1, lin1_w, lin1_b, k2, conv2,
           lin2_w, lin2_b):
    f32 = jnp.float32
    h = jnp.zeros((NPAD, D), f32).at[:N_NODES].set(x)
    pad = jnp.full((2, E_PAD - N_EDGES), N_NODES, jnp.int32)
    eip = jnp.concatenate([edge_index, pad], axis=1)
    src2 = eip[0].reshape(TOT_CH, CH)
    dst2 = eip[1].reshape(TOT_CH, CH)
    ones16 = jnp.ones((CH, 16), f32)
    zrow = jnp.zeros((ZR, DQ), f32)
    zcnt = jnp.zeros((ZR, 16), f32)
    ar = jnp.arange(D, dtype=jnp.int32)
    bpad = jnp.concatenate(
        [batch, jnp.full((NPAD - N_NODES,), NUM_GRAPHS, jnp.int32)])
    btiled = ((bpad[:, None] == (ar[None, :] // 10))
              & (ar[None, :] < 80)).astype(f32)
    k1f = k1.reshape(40, D)
    k1T = jnp.zeros((D, D), f32).at[:, :40].set(k1f.T)
    convrow = jnp.zeros((1, D), f32).at[0, :40].set(jnp.repeat(conv1, 10))
    w2T = jnp.zeros((D, D), f32).at[:, :16].set(lin2_w.T)
    b2 = jnp.zeros((1, D), f32).at[0, :16].set(lin2_b)

    zrow64 = jnp.zeros((ZR, DH), f32)
    inv = None
    for li, (Wl, bl, Wr) in enumerate(((W_l0, b_l0, W_r0),
                                       (W_l1, b_l1, W_r1),
                                       (W_l2, b_l2, W_r2))):
        if li == 0:
            hqs = [h[:, q * DQ:(q + 1) * DQ] for q in range(NQ)]
            *parts, cnt = _sc_aggregate0(hqs, src2, dst2, ones16, zrow, zcnt)
            inv = _tc_inv(cnt.reshape(NC, NPAD, 16))
            h = _tc_layer([p.reshape(NC, NPAD, DQ) for p in parts], inv, h,
                          Wl.T, bl.reshape(1, D), Wr.T)
        else:
            o0, o1 = _sc_aggregate_cs(h[:, :DH], h[:, DH:], src2, dst2,
                                      zrow64)
            h = _tc_layer_cs(o0, o1, inv, h, Wl.T, bl.reshape(1, D), Wr.T)
    y = _tc_mempool(h, btiled, k1T, convrow,
                    lin1_w.T, lin1_b.reshape(1, D), w2T, b2)
    return y[:NUM_GRAPHS, :16]


# speedup vs baseline: 1.0364x; 1.0364x over previous
import functools

import jax
import jax.numpy as jnp
from jax import lax
from jax.experimental import pallas as pl
from jax.experimental.pallas import tpu as pltpu
from jax.experimental.pallas import tpu_sc as plsc

N_NODES = 10000
N_EDGES = 320000
D = 128
NPAD = 10240
NUM_GRAPHS = 8
NEG = 0.01
NC = 2
NS = 16
NT = NC * NS
CH = 128
TOT_CH = 2560
NCHQ = TOT_CH // NT
NCHH = TOT_CH // NS
NBUF = 4
E_PAD = TOT_CH * CH
ROWS_TILE = NPAD // NS
NQ = 4
DQ = D // NQ
DH = D // 2
ZR = 128
BLK = 1024
GRID = NPAD // BLK


def _pipeline(h_hbm, src_v, dst_v, rbuf, gsem, acc_sp, nch,
              cnt_sp=None, ones_v=None):
    ngrp = nch // NBUF
    for p in range(NBUF):
        pltpu.async_copy(h_hbm.at[src_v.at[p]], rbuf[p], gsem[p])

    def grp(i, carry):
        for p in range(NBUF):
            j = i * NBUF + p
            pltpu.make_async_copy(
                h_hbm.at[pl.ds(0, CH)], rbuf[p], gsem[p]).wait()
            pltpu.sync_copy(rbuf[p], acc_sp.at[dst_v.at[j]], add=True)
            if cnt_sp is not None:
                pltpu.sync_copy(ones_v, cnt_sp.at[dst_v.at[j]], add=True)
            pltpu.async_copy(h_hbm.at[src_v.at[j + NBUF]], rbuf[p], gsem[p])
        return carry

    lax.fori_loop(0, ngrp - 1, grp, 0)
    for p in range(NBUF):
        j = (ngrp - 1) * NBUF + p
        pltpu.make_async_copy(
            h_hbm.at[pl.ds(0, CH)], rbuf[p], gsem[p]).wait()
        pltpu.sync_copy(rbuf[p], acc_sp.at[dst_v.at[j]], add=True)
        if cnt_sp is not None:
            pltpu.sync_copy(ones_v, cnt_sp.at[dst_v.at[j]], add=True)


def _sc_agg0_body(*refs):
    (h0, h1, h2, h3, src2_hbm, dst2_hbm, ones_hbm, zrow_hbm, zcnt_hbm,
     o0, o1, o2, o3, cnt_hbm,
     src_v, dst_v, r0, r1, r2, r3, ones_v, stage_v, zstage_v, czstage_v,
     cstage_v, acc_sp, cnt_sp,
     g0, g1, g2, g3) = refs
    hq = (h0, h1, h2, h3)
    oq = (o0, o1, o2, o3)
    rbuf = (r0, r1, r2, r3)
    gsem = (g0, g1, g2, g3)
    cid = lax.axis_index("c")
    sid = lax.axis_index("s")
    wid = cid * NS + sid
    tbase = sid * ROWS_TILE
    obase = cid * NPAD + tbase
    nz = ROWS_TILE // ZR
    pltpu.sync_copy(zrow_hbm, zstage_v)
    pltpu.sync_copy(ones_hbm, ones_v)
    pltpu.sync_copy(zcnt_hbm, czstage_v)
    pltpu.sync_copy(src2_hbm.at[pl.ds(wid * NCHQ, NCHQ)], src_v)
    pltpu.sync_copy(dst2_hbm.at[pl.ds(wid * NCHQ, NCHQ)], dst_v)
    for q in range(NQ):
        counts = q == 0
        for j in range(nz):
            pltpu.sync_copy(zstage_v, acc_sp.at[pl.ds(tbase + j * ZR, ZR)])
        if counts:
            for j in range(nz):
                pltpu.sync_copy(czstage_v,
                                cnt_sp.at[pl.ds(tbase + j * ZR, ZR)])
        plsc.subcore_barrier()
        _pipeline(hq[q], src_v, dst_v, rbuf, gsem, acc_sp, NCHQ,
                  cnt_sp if counts else None, ones_v)
        plsc.subcore_barrier()
        for j in range(nz):
            pltpu.sync_copy(acc_sp.at[pl.ds(tbase + j * ZR, ZR)], stage_v)
            pltpu.sync_copy(stage_v, oq[q].at[pl.ds(obase + j * ZR, ZR)])
        if counts:
            for j in range(nz):
                pltpu.sync_copy(cnt_sp.at[pl.ds(tbase + j * ZR, ZR)],
                                cstage_v)
                pltpu.sync_copy(cstage_v,
                                cnt_hbm.at[pl.ds(obase + j * ZR, ZR)])


def _sc_agg_cs_body(*refs):
    (hh0, hh1, src2_hbm, dst2_hbm, zrow_hbm,
     out0, out1,
     src_v, dst_v, r0, r1, r2, r3, stage_v, zstage_v, acc_sp,
     g0, g1, g2, g3) = refs
    rbuf = (r0, r1, r2, r3)
    gsem = (g0, g1, g2, g3)
    cid = lax.axis_index("c")
    sid = lax.axis_index("s")
    tbase = sid * ROWS_TILE
    nz = ROWS_TILE // ZR
    pltpu.sync_copy(zrow_hbm, zstage_v)
    pltpu.sync_copy(src2_hbm.at[pl.ds(sid * NCHH, NCHH)], src_v)
    pltpu.sync_copy(dst2_hbm.at[pl.ds(sid * NCHH, NCHH)], dst_v)
    for j in range(nz):
        pltpu.sync_copy(zstage_v, acc_sp.at[pl.ds(tbase + j * ZR, ZR)])
    plsc.subcore_barrier()

    @pl.when(cid == 0)
    def _():
        _pipeline(hh0, src_v, dst_v, rbuf, gsem, acc_sp, NCHH)

    @pl.when(cid == 1)
    def _():
        _pipeline(hh1, src_v, dst_v, rbuf, gsem, acc_sp, NCHH)

    plsc.subcore_barrier()

    @pl.when(cid == 0)
    def _():
        for j in range(nz):
            pltpu.sync_copy(acc_sp.at[pl.ds(tbase + j * ZR, ZR)], stage_v)
            pltpu.sync_copy(stage_v, out0.at[pl.ds(tbase + j * ZR, ZR)])

    @pl.when(cid == 1)
    def _():
        for j in range(nz):
            pltpu.sync_copy(acc_sp.at[pl.ds(tbase + j * ZR, ZR)], stage_v)
            pltpu.sync_copy(stage_v, out1.at[pl.ds(tbase + j * ZR, ZR)])


@functools.cache
def _get_sc_kernels():
    mesh = plsc.VectorSubcoreMesh(core_axis_name="c", subcore_axis_name="s",
                                  num_cores=NC, num_subcores=NS)
    qout = [jax.ShapeDtypeStruct((NC * NPAD, DQ), jnp.float32)
            for _ in range(NQ)]
    cparams = pltpu.CompilerParams(use_tc_tiling_on_sc=False)
    sc_agg0 = pl.kernel(
        _sc_agg0_body,
        out_type=qout + [jax.ShapeDtypeStruct((NC * NPAD, 16), jnp.float32)],
        compiler_params=cparams,
        mesh=mesh,
        scratch_types=[pltpu.VMEM((NCHQ, CH), jnp.int32),
                       pltpu.VMEM((NCHQ, CH), jnp.int32)]
                      + [pltpu.VMEM((CH, DQ), jnp.float32)
                         for _ in range(NBUF)]
                      + [pltpu.VMEM((CH, 16), jnp.float32),
                         pltpu.VMEM((ZR, DQ), jnp.float32),
                         pltpu.VMEM((ZR, DQ), jnp.float32),
                         pltpu.VMEM((ZR, 16), jnp.float32),
                         pltpu.VMEM((ZR, 16), jnp.float32),
                         pltpu.VMEM_SHARED((NPAD, DQ), jnp.float32),
                         pltpu.VMEM_SHARED((NPAD, 16), jnp.float32)]
                      + [pltpu.SemaphoreType.DMA for _ in range(NBUF)],
    )
    sc_agg_cs = pl.kernel(
        _sc_agg_cs_body,
        out_type=[jax.ShapeDtypeStruct((NPAD, DH), jnp.float32),
                  jax.ShapeDtypeStruct((NPAD, DH), jnp.float32)],
        compiler_params=cparams,
        mesh=mesh,
        scratch_types=[pltpu.VMEM((NCHH, CH), jnp.int32),
                       pltpu.VMEM((NCHH, CH), jnp.int32)]
                      + [pltpu.VMEM((CH, DH), jnp.float32)
                         for _ in range(NBUF)]
                      + [pltpu.VMEM((ZR, DH), jnp.float32),
                         pltpu.VMEM((ZR, DH), jnp.float32),
                         pltpu.VMEM_SHARED((NPAD, DH), jnp.float32)]
                      + [pltpu.SemaphoreType.DMA for _ in range(NBUF)],
    )
    return sc_agg0, sc_agg_cs


def _sc_aggregate0(hqs, src2, dst2, ones16, zrow, zcnt):
    return _get_sc_kernels()[0](*hqs, src2, dst2, ones16, zrow, zcnt)


def _sc_aggregate_cs(hh0, hh1, src2, dst2, zrow64):
    return _get_sc_kernels()[1](hh0, hh1, src2, dst2, zrow64)


def _tc_inv_body(cnt_ref, inv_ref):
    c = cnt_ref[0, :, 0:1] + cnt_ref[1, :, 0:1]
    inv = 1.0 / jnp.maximum(c, 1.0)
    inv_ref[...] = jnp.broadcast_to(inv, (BLK, D))


def _tc_inv(cnt):
    return pl.pallas_call(
        _tc_inv_body,
        grid=(GRID,),
        in_specs=[pl.BlockSpec((2, BLK, 16), lambda i: (0, i, 0))],
        out_specs=pl.BlockSpec((BLK, D), lambda i: (i, 0)),
        out_shape=jax.ShapeDtypeStruct((NPAD, D), jnp.float32),
    )(cnt)


def _tc_layer_body(p0_ref, p1_ref, p2_ref, p3_ref, inv_ref, h_ref,
                   wlT_ref, bl_ref, wrT_ref, out_ref):
    agg = jnp.concatenate(
        [p[0] + p[1] for p in (p0_ref, p1_ref, p2_ref, p3_ref)],
        axis=-1) * inv_ref[...]
    o = (jnp.dot(agg, wlT_ref[...], preferred_element_type=jnp.float32)
         + jnp.dot(h_ref[...], wrT_ref[...], preferred_element_type=jnp.float32)
         + bl_ref[...])
    out_ref[...] = jnp.where(o >= 0, o, NEG * o)


def _tc_layer(parts, inv, h, wlT, bl, wrT):
    qspec = pl.BlockSpec((2, BLK, DQ), lambda i: (0, i, 0))
    return pl.pallas_call(
        _tc_layer_body,
        grid=(GRID,),
        in_specs=[qspec, qspec, qspec, qspec,
                  pl.BlockSpec((BLK, D), lambda i: (i, 0)),
                  pl.BlockSpec((BLK, D), lambda i: (i, 0)),
                  pl.BlockSpec((D, D), lambda i: (0, 0)),
                  pl.BlockSpec((1, D), lambda i: (0, 0)),
                  pl.BlockSpec((D, D), lambda i: (0, 0))],
        out_specs=pl.BlockSpec((BLK, D), lambda i: (i, 0)),
        out_shape=jax.ShapeDtypeStruct((NPAD, D), jnp.float32),
    )(*parts, inv, h, wlT, bl, wrT)


def _tc_layer_cs_body(o0_ref, o1_ref, inv_ref, h_ref,
                      wlT_ref, bl_ref, wrT_ref, out_ref):
    agg = jnp.concatenate([o0_ref[...], o1_ref[...]], axis=-1) * inv_ref[...]
    o = (jnp.dot(agg, wlT_ref[...], preferred_element_type=jnp.float32)
         + jnp.dot(h_ref[...], wrT_ref[...], preferred_element_type=jnp.float32)
         + bl_ref[...])
    out_ref[...] = jnp.where(o >= 0, o, NEG * o)


def _tc_layer_cs(o0, o1, inv, h, wlT, bl, wrT):
    hspec = pl.BlockSpec((BLK, DH), lambda i: (i, 0))
    return pl.pallas_call(
        _tc_layer_cs_body,
        grid=(GRID,),
        in_specs=[hspec, hspec,
                  pl.BlockSpec((BLK, D), lambda i: (i, 0)),
                  pl.BlockSpec((BLK, D), lambda i: (i, 0)),
                  pl.BlockSpec((D, D), lambda i: (0, 0)),
                  pl.BlockSpec((1, D), lambda i: (0, 0)),
                  pl.BlockSpec((D, D), lambda i: (0, 0))],
        out_specs=pl.BlockSpec((BLK, D), lambda i: (i, 0)),
        out_shape=jax.ShapeDtypeStruct((NPAD, D), jnp.float32),
    )(o0, o1, inv, h, wlT, bl, wrT)


def _mempool_body(h_ref, bt_ref, k1T_ref, conv_ref, w1T_ref, b1_ref,
                  w2T_ref, b2_ref, out_ref, acc_ref):
    i = pl.program_id(0)
    h = h_ref[...]
    k1T = k1T_ref[...]
    xk = jnp.dot(h, k1T, preferred_element_type=jnp.float32)
    x2 = jnp.sum(h * h, axis=1, keepdims=True)
    k2 = jnp.sum(k1T * k1T, axis=0, keepdims=True)
    col = lax.broadcasted_iota(jnp.int32, (BLK, D), 1)
    dist = jnp.maximum(x2 + k2 - 2.0 * xk, 0.0)
    t = 1.0 / (1.0 + dist)
    r = lax.broadcasted_iota(jnp.int32, (D, D), 0)
    c2 = lax.broadcasted_iota(jnp.int32, (D, D), 1)
    G = jnp.where((r // 10 == c2 // 10) & (r < 40) & (c2 < 40), 1.0, 0.0)
    gsum = jnp.dot(t, G, preferred_element_type=jnp.float32)
    u = jnp.where(col < 40, (t / gsum) * conv_ref[...], 0.0)
    P = jnp.where((r % 10 == c2) & (r < 40) & (c2 < 10), 1.0, 0.0)
    logits = jnp.dot(u, P, preferred_element_type=jnp.float32)
    neg = jnp.float32(-1e30)
    lm = jnp.max(jnp.where(col < 10, logits, neg), axis=1, keepdims=True)
    e = jnp.where(col < 10, jnp.exp(logits - lm), 0.0)
    S = e / jnp.sum(e, axis=1, keepdims=True)
    T = jnp.where((r < 10) & (r == c2 % 10), 1.0, 0.0)
    W = jnp.dot(S, T, preferred_element_type=jnp.float32) * bt_ref[...]
    contrib = lax.dot_general(W, h, (((0,), (0,)), ((), ())),
                              preferred_element_type=jnp.float32)

    @pl.when(i == 0)
    def _():
        acc_ref[...] = contrib

    @pl.when(i > 0)
    def _():
        acc_ref[...] += contrib

    @pl.when(i == GRID - 1)
    def _():
        acc = acc_ref[...]
        xp = jnp.dot(acc, w1T_ref[...], preferred_element_type=jnp.float32) + b1_ref[...]
        xp = jnp.where(xp >= 0, xp, NEG * xp)
        Q = jnp.where((c2 // 10 == r) & (r < 8) & (c2 < 80), 1.0, 0.0)
        ypre = jnp.dot(Q, xp, preferred_element_type=jnp.float32)
        out_ref[...] = (jnp.dot(ypre, w2T_ref[...], preferred_element_type=jnp.float32)
                        + b2_ref[...])


def _tc_mempool(h, btiled, k1T, convrow, w1T, b1, w2T, b2):
    return pl.pallas_call(
        _mempool_body,
        grid=(GRID,),
        in_specs=[pl.BlockSpec((BLK, D), lambda i: (i, 0)),
                  pl.BlockSpec((BLK, D), lambda i: (i, 0)),
                  pl.BlockSpec((D, D), lambda i: (0, 0)),
                  pl.BlockSpec((1, D), lambda i: (0, 0)),
                  pl.BlockSpec((D, D), lambda i: (0, 0)),
                  pl.BlockSpec((1, D), lambda i: (0, 0)),
                  pl.BlockSpec((D, D), lambda i: (0, 0)),
                  pl.BlockSpec((1, D), lambda i: (0, 0))],
        out_specs=pl.BlockSpec((D, D), lambda i: (0, 0)),
        out_shape=jax.ShapeDtypeStruct((D, D), jnp.float32),
        scratch_shapes=[pltpu.VMEM((D, D), jnp.float32)],
    )(h, btiled, k1T, convrow, w1T, b1, w2T, b2)


def kernel(x, edge_index, batch, W_l0, b_l0, W_r0, W_l1, b_l1, W_r1,
           W_l2, b_l2, W_r2, k1, conv1, lin1_w, lin1_b, k2, conv2,
           lin2_w, lin2_b):
    f32 = jnp.float32
    h = jnp.zeros((NPAD, D), f32).at[:N_NODES].set(x)
    pad = jnp.full((2, E_PAD - N_EDGES), N_NODES, jnp.int32)
    eip = jnp.concatenate([edge_index, pad], axis=1)
    src2 = eip[0].reshape(TOT_CH, CH)
    dst2 = eip[1].reshape(TOT_CH, CH)
    ones16 = jnp.ones((CH, 16), f32)
    zrow = jnp.zeros((ZR, DQ), f32)
    zcnt = jnp.zeros((ZR, 16), f32)
    ar = jnp.arange(D, dtype=jnp.int32)
    bpad = jnp.concatenate(
        [batch, jnp.full((NPAD - N_NODES,), NUM_GRAPHS, jnp.int32)])
    btiled = ((bpad[:, None] == (ar[None, :] // 10))
              & (ar[None, :] < 80)).astype(f32)
    k1f = k1.reshape(40, D)
    k1T = jnp.zeros((D, D), f32).at[:, :40].set(k1f.T)
    convrow = jnp.zeros((1, D), f32).at[0, :40].set(jnp.repeat(conv1, 10))
    w2T = jnp.zeros((D, D), f32).at[:, :16].set(lin2_w.T)
    b2 = jnp.zeros((1, D), f32).at[0, :16].set(lin2_b)

    zrow64 = jnp.zeros((ZR, DH), f32)
    inv = None
    for li, (Wl, bl, Wr) in enumerate(((W_l0, b_l0, W_r0),
                                       (W_l1, b_l1, W_r1),
                                       (W_l2, b_l2, W_r2))):
        if li == 0:
            hqs = [h[:, q * DQ:(q + 1) * DQ] for q in range(NQ)]
            *parts, cnt = _sc_aggregate0(hqs, src2, dst2, ones16, zrow, zcnt)
            inv = _tc_inv(cnt.reshape(NC, NPAD, 16))
            h = _tc_layer([p.reshape(NC, NPAD, DQ) for p in parts], inv, h,
                          Wl.T, bl.reshape(1, D), Wr.T)
        else:
            o0, o1 = _sc_aggregate_cs(h[:, :DH], h[:, DH:], src2, dst2,
                                      zrow64)
            h = _tc_layer_cs(o0, o1, inv, h, Wl.T, bl.reshape(1, D), Wr.T)
    y = _tc_mempool(h, btiled, k1T, convrow,
                    lin1_w.T, lin1_b.reshape(1, D), w2T, b2)
    return y[:NUM_GRAPHS, :16]
